# baseline (speedup 1.0000x reference)
import functools
import jax
import jax.numpy as jnp
from jax import lax
from jax.experimental import pallas as pl
from jax.experimental.pallas import tpu as pltpu
from jax.experimental.pallas import tpu_sc as plsc

_N = 10000
_E = 320000
_H = 64
_P = 5
_NC = 2
_NS = 16
_NW = _NC * _NS
_CE = 16000
_NCH = _E // _CE
_EP = _E // _NW

_mesh = plsc.VectorSubcoreMesh(core_axis_name="c", subcore_axis_name="s")
_sc_params = pltpu.CompilerParams(needs_layout_passes=False)


def _bf16_pack_pair(v0, v1):
    u0 = plsc.bitcast(v0, jnp.int32)
    u1 = plsc.bitcast(v1, jnp.int32)
    hi = jnp.bitwise_and(u0 + 0x8000, jnp.int32(-65536))
    lo = lax.shift_right_logical(u1 + 0x8000, 16)
    return jnp.bitwise_or(hi, lo)


def _bf16_hi(w):
    return plsc.bitcast(jnp.bitwise_and(w, jnp.int32(-65536)), jnp.float32)


def _bf16_lo(w):
    return plsc.bitcast(lax.shift_left(w, 16), jnp.float32)


def _zero_f32(buf, nwords):
    @plsc.parallel_loop(0, nwords, 16, unroll=4)
    def zb(i):
        buf[pl.ds(i, 16)] = jnp.zeros((16,), jnp.float32)


@functools.partial(
    pl.kernel,
    out_type=jax.ShapeDtypeStruct((_NW * _N,), jnp.float32),
    mesh=_mesh,
    scratch_types=[
        pltpu.VMEM((_N,), jnp.float32),
        pltpu.VMEM((_EP,), jnp.int32),
        pltpu.VMEM((_EP,), jnp.float32),
    ],
    compiler_params=_sc_params,
)
def _sc_deg(col_hbm, ew_hbm, out_hbm, dacc, cb, wb):
    wid = lax.axis_index("c") * _NS + lax.axis_index("s")
    base = wid * _EP
    pltpu.sync_copy(col_hbm.at[pl.ds(base, _EP)], cb)
    pltpu.sync_copy(ew_hbm.at[pl.ds(base, _EP)], wb)
    _zero_f32(dacc, _N)

    @plsc.parallel_loop(0, _EP, 16, unroll=8)
    def body(i):
        s = pl.ds(i, 16)
        plsc.addupdate_scatter(dacc, [cb[s]], wb[s])
    pltpu.sync_copy(dacc, out_hbm.at[pl.ds(wid * _N, _N)])


_EP2 = _E // 16


@functools.partial(
    pl.kernel,
    out_type=(
        jax.ShapeDtypeStruct((_E // 2,), jnp.int32),
        jax.ShapeDtypeStruct((_E,), jnp.int32),
    ),
    mesh=_mesh,
    scratch_types=[
        pltpu.VMEM((_N,), jnp.float32),
        pltpu.VMEM((_EP2,), jnp.int32),
        pltpu.VMEM((_EP2,), jnp.int32),
        pltpu.VMEM((_EP2,), jnp.float32),
        pltpu.VMEM((_EP2 // 2,), jnp.int32),
        pltpu.VMEM((_EP2,), jnp.int32),
    ],
    compiler_params=_sc_params,
)
def _sc_norm(row_hbm, col_hbm, ew_hbm, dis_hbm, nrm_hbm, pk_hbm,
             disv, rb, cb, wb, on, op):
    cid = lax.axis_index("c")
    sid = lax.axis_index("s")
    wid = cid * _NS + sid

    @pl.when(wid < 16)
    def _():
        base = wid * _EP2
        pltpu.sync_copy(dis_hbm, disv)
        pltpu.sync_copy(row_hbm.at[pl.ds(base, _EP2)], rb)
        pltpu.sync_copy(col_hbm.at[pl.ds(base, _EP2)], cb)
        pltpu.sync_copy(ew_hbm.at[pl.ds(base, _EP2)], wb)

        @plsc.parallel_loop(0, _EP2, 32, unroll=8)
        def body(i):
            sa = pl.ds(i, 16)
            sb = pl.ds(i + 16, 16)
            ra = rb[sa]
            ca = cb[sa]
            na = plsc.load_gather(disv, [ra]) * wb[sa] \
                * plsc.load_gather(disv, [ca])
            rbb = rb[sb]
            cbb = cb[sb]
            nb_ = plsc.load_gather(disv, [rbb]) * wb[sb] \
                * plsc.load_gather(disv, [cbb])
            on[pl.ds(pl.multiple_of(lax.shift_right_logical(i, 1), 16), 16)] \
                = _bf16_pack_pair(na, nb_)
            op[sa] = lax.shift_left(ra, 14) + ca
            op[sb] = lax.shift_left(rbb, 14) + cbb
        pltpu.sync_copy(on, nrm_hbm.at[pl.ds(wid * (_EP2 // 2), _EP2 // 2)])
        pltpu.sync_copy(op, pk_hbm.at[pl.ds(base, _EP2)])


@functools.partial(
    pl.kernel,
    out_type=jax.ShapeDtypeStruct((_H * _N,), jnp.float32),
    mesh=_mesh,
    scratch_types=[
        pltpu.VMEM((2 * _N,), jnp.float32),
        pltpu.VMEM((2 * _N,), jnp.float32),
        pltpu.VMEM((2 * _N,), jnp.float32),
        pltpu.VMEM((_N,), jnp.int32),
        pltpu.VMEM((_CE,), jnp.int32),
        pltpu.VMEM((_CE,), jnp.int32),
        pltpu.VMEM((_CE // 2,), jnp.int32),
        pltpu.VMEM((_CE // 2,), jnp.int32),
        pltpu.VMEM((128,), jnp.float32),
        pltpu.SemaphoreType.DMA,
        pltpu.SemaphoreType.DMA,
        pltpu.SemaphoreType.DMA,
        pltpu.SemaphoreType.DMA,
    ],
    compiler_params=_sc_params,
)
def _sc_layer(h_hbm, pk_hbm, nm_hbm, cf_hbm, out_hbm,
              tx0, tx1, acc, packb, pb0, pb1, nb0, nb1, cfs,
              sp0, sp1, sn0, sn1):

    wid = lax.axis_index("c") * _NS + lax.axis_index("s")
    hbase = wid * (2 * _N)
    pltpu.sync_copy(cf_hbm, cfs)
    pltpu.sync_copy(h_hbm.at[pl.ds(hbase, 2 * _N)], tx0)

    c0v = cfs[pl.ds(0, 16)] + 1.0
    tx0_hi = tx0.at[pl.ds(_N, _N)]
    tx1_hi = tx1.at[pl.ds(_N, _N)]
    acc_hi = acc.at[pl.ds(_N, _N)]

    @plsc.parallel_loop(0, _N, 16, unroll=4)
    def init_acc(i):
        s = pl.ds(i, 16)
        v0 = tx0[s]
        v1 = tx0_hi[s]
        acc[s] = v0 * c0v
        acc_hi[s] = v1 * c0v
        packb[s] = _bf16_pack_pair(v0, v1)
        tx1[s] = jnp.zeros((16,), jnp.float32)
        tx1_hi[s] = jnp.zeros((16,), jnp.float32)

    pbufs = (pb0, pb1)
    nbufs = (nb0, nb1)
    psems = (sp0, sp1)
    nsems = (sn0, sn1)

    def start_chunk(c, par):
        pltpu.make_async_copy(
            pk_hbm.at[pl.ds(c * _CE, _CE)], pbufs[par], psems[par]).start()
        pltpu.make_async_copy(
            nm_hbm.at[pl.ds(c * (_CE // 2), _CE // 2)],
            nbufs[par], nsems[par]).start()

    def wait_chunk(c, par):
        pltpu.make_async_copy(
            pk_hbm.at[pl.ds(c * _CE, _CE)], pbufs[par], psems[par]).wait()
        pltpu.make_async_copy(
            nm_hbm.at[pl.ds(c * (_CE // 2), _CE // 2)],
            nbufs[par], nsems[par]).wait()

    for k in range(1, _P + 1):
        cur = tx0 if k % 2 == 1 else tx1
        nxt = tx1 if k % 2 == 1 else tx0
        cur_hi = cur.at[pl.ds(_N, _N)]
        nxt_hi = nxt.at[pl.ds(_N, _N)]
        start_chunk(0, 0)

        def grp(pb, nb, base):
            nw = nb[pl.ds(pl.multiple_of(lax.shift_right_logical(base, 1), 16),
                          16)]
            for half in (0, 1):
                s = pl.ds(base + 16 * half, 16)
                p16 = pb[s]
                nm16 = _bf16_hi(nw) if half == 0 else _bf16_lo(nw)
                c16 = jnp.bitwise_and(p16, 16383)
                r16 = lax.shift_right_logical(p16, 14)
                w = plsc.load_gather(packb, [c16])
                plsc.addupdate_scatter(nxt, [r16], _bf16_hi(w) * nm16)
                plsc.addupdate_scatter(nxt_hi, [r16], _bf16_lo(w) * nm16)

        def chunk_pair(j, c):
            for par in (0, 1):
                ch = 2 * j + par

                @pl.when(ch + 1 < _NCH)
                def _():
                    start_chunk(ch + 1, 1 - par)

                wait_chunk(ch, par)

                @plsc.parallel_loop(0, _CE, 32, unroll=5)
                def inner(base):
                    grp(pbufs[par], nbufs[par], base)
            return c
        lax.fori_loop(0, _NCH // 2, chunk_pair, 0)

        ckv = cfs[pl.ds(k * 16, 16)]

        @plsc.parallel_loop(0, _N, 16, unroll=4)
        def upd_acc(i):
            s = pl.ds(i, 16)
            v0 = nxt[s]
            v1 = nxt_hi[s]
            acc[s] = acc[s] + v0 * ckv
            acc_hi[s] = acc_hi[s] + v1 * ckv
            if k < _P:
                packb[s] = _bf16_pack_pair(v0, v1)
                cur[s] = jnp.zeros((16,), jnp.float32)
                cur_hi[s] = jnp.zeros((16,), jnp.float32)

    pltpu.sync_copy(acc, out_hbm.at[pl.ds(hbase, 2 * _N)])


def _make_sc_edgez(eh):
    nch = eh // _CE

    @functools.partial(
        pl.kernel,
        out_type=jax.ShapeDtypeStruct(((_H // 2) * eh,), jnp.int32),
        mesh=_mesh,
        scratch_types=[
            pltpu.VMEM((2 * _N,), jnp.float32),
            pltpu.VMEM((2 * _N,), jnp.float32),
            pltpu.VMEM((_N,), jnp.int32),
            pltpu.VMEM((_N,), jnp.int32),
            pltpu.VMEM((_CE,), jnp.int32),
            pltpu.VMEM((_CE,), jnp.int32),
            pltpu.VMEM((_CE,), jnp.int32),
            pltpu.VMEM((_CE,), jnp.int32),
            pltpu.SemaphoreType.DMA,
            pltpu.SemaphoreType.DMA,
            pltpu.SemaphoreType.DMA,
            pltpu.SemaphoreType.DMA,
        ],
        compiler_params=_sc_params,
    )
    def _sc_edgez(wr_hbm, wc_hbm, pk_hbm, z_hbm,
                  wrv, wcv, wrp, wcp, pb0, pb1, zb0, zb1,
                  sp0, sp1, so0, so1):
        wid = lax.axis_index("c") * _NS + lax.axis_index("s")
        fr0 = 2 * wid
        pltpu.sync_copy(wr_hbm.at[pl.ds(fr0 * _N, 2 * _N)], wrv)
        pltpu.sync_copy(wc_hbm.at[pl.ds(fr0 * _N, 2 * _N)], wcv)
        wrv_hi = wrv.at[pl.ds(_N, _N)]
        wcv_hi = wcv.at[pl.ds(_N, _N)]

        @plsc.parallel_loop(0, _N, 16, unroll=4)
        def pack_src(i):
            s = pl.ds(i, 16)
            wrp[s] = _bf16_pack_pair(wrv[s], wrv_hi[s])
            wcp[s] = _bf16_pack_pair(wcv[s], wcv_hi[s])

        pbufs = (pb0, pb1)
        zbufs = (zb0, zb1)
        psems = (sp0, sp1)
        osems = (so0, so1)

        def start_in(c, par):
            pltpu.make_async_copy(
                pk_hbm.at[pl.ds(c * _CE, _CE)], pbufs[par], psems[par]).start()

        def wait_in(c, par):
            pltpu.make_async_copy(
                pk_hbm.at[pl.ds(c * _CE, _CE)], pbufs[par], psems[par]).wait()

        def start_out(c, par):
            pltpu.make_async_copy(
                zbufs[par],
                z_hbm.at[pl.ds(wid * eh + c * _CE, _CE)], osems[par]).start()

        def wait_out(c, par):
            pltpu.make_async_copy(
                zbufs[par],
                z_hbm.at[pl.ds(wid * eh + c * _CE, _CE)], osems[par]).wait()

        start_in(0, 0)

        def chunk_pair(j, c):
            for par in (0, 1):
                ch = 2 * j + par

                @pl.when(ch + 1 < nch)
                def _():
                    start_in(ch + 1, 1 - par)

                wait_in(ch, par)
                @pl.when(ch >= 2)
                def _():
                    wait_out(ch - 2, par)

                pb = pbufs[par]
                zb = zbufs[par]

                @plsc.parallel_loop(0, _CE, 16, unroll=10)
                def inner(base):
                    s = pl.ds(base, 16)
                    p16 = pb[s]
                    c16 = jnp.bitwise_and(p16, 16383)
                    r16 = lax.shift_right_logical(p16, 14)
                    gr = plsc.load_gather(wrp, [r16])
                    gc = plsc.load_gather(wcp, [c16])
                    f0 = _bf16_hi(gr) + _bf16_hi(gc)
                    f1 = _bf16_lo(gr) + _bf16_lo(gc)
                    zb[s] = _bf16_pack_pair(f0, f1)
                start_out(ch, par)
            return c
        lax.fori_loop(0, nch // 2, chunk_pair, 0)
        wait_out(nch - 2, 0)
        wait_out(nch - 1, 1)

    return _sc_edgez


_sc_edgez_half = _make_sc_edgez(_E // 2)


def _stats_coeffs(hT, cw1a, cw1b, cb1l, cw2p, cb2p):
    xm = jnp.mean(hT, axis=1, keepdims=True)
    m = jnp.mean(hT)
    sq = jnp.sum(hT * hT)
    mm = jnp.float32(_N * _H)
    var1 = (sq - mm * m * m) / (mm - 1.0)
    st = jnp.sqrt(var1)
    stats = jnp.concatenate(
        [jnp.full((1, 1), 1.0, jnp.float32) * m,
         jnp.full((1, 1), 1.0, jnp.float32) * st,
         jnp.full((1, 1), float(_N), jnp.float32),
         jnp.full((1, 1), float(_E), jnp.float32)], axis=0)
    hid = (lax.dot_general(xm, cw1a, (((0,), (0,)), ((), ())))
           + lax.dot_general(stats, cw1b, (((0,), (0,)), ((), ())))
           + cb1l)
    hid = jnp.maximum(hid, 0.0)
    lo = lax.dot_general(hid, cw2p, (((1,), (0,)), ((), ()))) + cb2p
    z = lo - jnp.max(lo, axis=1, keepdims=True)
    ez = jnp.exp(z)
    return ez / jnp.sum(ez, axis=1, keepdims=True)


def _tc_prep_body(x_ref, w_ref, b_ref, dp_ref, cw1a, cw1b, cb1l, cw2p, cb2p,
                  hT_ref, dis_ref, cf_ref):
    hT = lax.dot_general(w_ref[...], x_ref[...], (((0,), (1,)), ((), ())))
    hT = hT + b_ref[...]
    hT_ref[...] = hT
    deg = jnp.sum(dp_ref[...], axis=0, keepdims=True)
    dis = jnp.minimum(lax.rsqrt(deg), 1e6)
    dis = jnp.where(jnp.isinf(dis), 0.0, dis)
    dis_ref[...] = dis
    cf_ref[...] = _stats_coeffs(hT, cw1a[...], cw1b[...], cb1l[...],
                                cw2p[...], cb2p[...])


_tc_prep = pl.pallas_call(
    _tc_prep_body,
    out_shape=(
        jax.ShapeDtypeStruct((_H, _N), jnp.float32),
        jax.ShapeDtypeStruct((1, _N), jnp.float32),
        jax.ShapeDtypeStruct((1, 8), jnp.float32),
    ),
)


def _ln(hrT, g, b):
    mu = jnp.mean(hrT, axis=0, keepdims=True)
    d = hrT - mu
    var = jnp.mean(d * d, axis=0, keepdims=True)
    return d * lax.rsqrt(var + 1e-5) * g + b


def _tc_inter_body(hr_ref, g_ref, b_ref, cw1a, cw1b, cb1l, cw2p, cb2p,
                   hn_ref, cf_ref):
    hn = _ln(hr_ref[...], g_ref[...], b_ref[...])
    hn_ref[...] = hn
    cf_ref[...] = _stats_coeffs(hn, cw1a[...], cw1b[...], cb1l[...],
                                cw2p[...], cb2p[...])


_tc_inter = pl.pallas_call(
    _tc_inter_body,
    out_shape=(
        jax.ShapeDtypeStruct((_H, _N), jnp.float32),
        jax.ShapeDtypeStruct((1, 8), jnp.float32),
    ),
)


def _tc_final_body(hr_ref, g_ref, b_ref, w1a_ref, w1b_ref,
                   hn_ref, wr_ref, wc_ref):
    hn = _ln(hr_ref[...], g_ref[...], b_ref[...])
    hn_ref[...] = hn
    wr_ref[...] = lax.dot_general(w1a_ref[...], hn, (((0,), (0,)), ((), ())))
    wc_ref[...] = lax.dot_general(w1b_ref[...], hn, (((0,), (0,)), ((), ())))


_tc_final = pl.pallas_call(
    _tc_final_body,
    out_shape=(
        jax.ShapeDtypeStruct((_H, _N), jnp.float32),
        jax.ShapeDtypeStruct((_H, _N), jnp.float32),
        jax.ShapeDtypeStruct((_H, _N), jnp.float32),
    ),
)

_EB = 3200


def _tc_edge_body(z_ref, eb1e_ref, eb1o_ref, w2e_ref, w2o_ref, eb2_ref,
                  w3_ref, eb3_ref, out_ref):
    w = z_ref[...]
    ze = lax.bitcast_convert_type(
        jnp.bitwise_and(w, jnp.int32(-65536)), jnp.float32)
    zo = lax.bitcast_convert_type(
        lax.shift_left(w, 16), jnp.float32)
    a1e = jnp.maximum(ze + eb1e_ref[...], 0.0)
    a1o = jnp.maximum(zo + eb1o_ref[...], 0.0)
    a2 = (lax.dot_general(w2e_ref[...], a1e, (((0,), (0,)), ((), ())))
          + lax.dot_general(w2o_ref[...], a1o, (((0,), (0,)), ((), ()))))
    a2 = jnp.maximum(a2 + eb2_ref[...], 0.0)
    s = lax.dot_general(w3_ref[...], a2, (((0,), (0,)), ((), ())))
    s = s + eb3_ref[...]
    out_ref[...] = 1.0 / (1.0 + jnp.exp(-s))


def _make_tc_edge(eh):
    return pl.pallas_call(
        _tc_edge_body,
        grid=(eh // _EB,),
        in_specs=[
            pl.BlockSpec((_H // 2, _EB), lambda i: (0, i)),
            pl.BlockSpec((_H // 2, 1), lambda i: (0, 0)),
            pl.BlockSpec((_H // 2, 1), lambda i: (0, 0)),
            pl.BlockSpec((32, 32), lambda i: (0, 0)),
            pl.BlockSpec((32, 32), lambda i: (0, 0)),
            pl.BlockSpec((32, 1), lambda i: (0, 0)),
            pl.BlockSpec((32, 1), lambda i: (0, 0)),
            pl.BlockSpec((1, 1), lambda i: (0, 0)),
        ],
        out_specs=pl.BlockSpec((1, _EB), lambda i: (0, i)),
        out_shape=jax.ShapeDtypeStruct((1, eh), jnp.float32),
    )


_tc_edge_half = _make_tc_edge(_E // 2)


def kernel(x, edge_index, edge_weight, in_w, in_b, cw1, cb1, cw2, cb2,
           ln_g, ln_b, ew1, eb1, ew2, eb2, ew3, eb3):
    L = cw1.shape[0]
    row = edge_index[0]
    col = edge_index[1]

    in_b2 = in_b.reshape(_H, 1)
    cw1a = [cw1[l, :_H, :] for l in range(L)]
    cw1b = [cw1[l, _H:, :] for l in range(L)]
    cb1l = [cb1[l].reshape(1, -1) for l in range(L)]
    cw2p = [jnp.pad(cw2[l], ((0, 0), (0, 8 - (_P + 1)))) for l in range(L)]
    cb2p = [jnp.pad(cb2[l], (0, 8 - (_P + 1)),
                    constant_values=-1e30).reshape(1, 8) for l in range(L)]

    deg_parts = _sc_deg(col, edge_weight).reshape(_NW, _N)
    hT, dis, cf = _tc_prep(x, in_w, in_b2, deg_parts,
                           cw1a[0], cw1b[0], cb1l[0], cw2p[0], cb2p[0])
    nrm, packed = _sc_norm(row, col, edge_weight, dis.reshape(_N))

    for l in range(L):
        cf_rep = jnp.broadcast_to(cf.reshape(8)[:, None], (8, 16)).reshape(128)
        hr_flat = _sc_layer(hT.reshape(-1), packed, nrm, cf_rep)
        hrT = hr_flat.reshape(_H, _N)
        if l < L - 1:
            hT, cf = _tc_inter(hrT, ln_g[l].reshape(_H, 1),
                               ln_b[l].reshape(_H, 1),
                               cw1a[l + 1], cw1b[l + 1], cb1l[l + 1],
                               cw2p[l + 1], cb2p[l + 1])
        else:
            hT, wrT, wcT = _tc_final(hrT, ln_g[l].reshape(_H, 1),
                                     ln_b[l].reshape(_H, 1),
                                     ew1[:_H, :], ew1[_H:, :])

    wr_flat = wrT.reshape(-1)
    wc_flat = wcT.reshape(-1)
    eh = _E // 2
    mlp_args = (eb1[0::2].reshape(_H // 2, 1),
                eb1[1::2].reshape(_H // 2, 1),
                ew2[0::2, :], ew2[1::2, :],
                eb2.reshape(32, 1),
                ew3, eb3.reshape(1, 1))
    zw_a = _sc_edgez_half(wr_flat, wc_flat, packed[:eh])
    preds_a = _tc_edge_half(zw_a.reshape(_H // 2, eh), *mlp_args)
    zw_b = _sc_edgez_half(wr_flat, wc_flat, packed[eh:])
    preds_b = _tc_edge_half(zw_b.reshape(_H // 2, eh), *mlp_args)
    preds = jnp.concatenate([preds_a.reshape(eh), preds_b.reshape(eh)])
    return preds, hT.T


# speedup vs baseline: 1.0083x; 1.0083x over previous
import functools
import jax
import jax.numpy as jnp
from jax import lax
from jax.experimental import pallas as pl
from jax.experimental.pallas import tpu as pltpu
from jax.experimental.pallas import tpu_sc as plsc

_N = 10000
_E = 320000
_H = 64
_P = 5
_NC = 2
_NS = 16
_NW = _NC * _NS
_CE = 8000
_NCH = _E // _CE
_EP = _E // _NW

_mesh = plsc.VectorSubcoreMesh(core_axis_name="c", subcore_axis_name="s")
_sc_params = pltpu.CompilerParams(needs_layout_passes=False)


def _bf16_pack_pair(v0, v1):
    u0 = plsc.bitcast(v0, jnp.int32)
    u1 = plsc.bitcast(v1, jnp.int32)
    hi = jnp.bitwise_and(u0 + 0x8000, jnp.int32(-65536))
    lo = lax.shift_right_logical(u1 + 0x8000, 16)
    return jnp.bitwise_or(hi, lo)


def _bf16_hi(w):
    return plsc.bitcast(jnp.bitwise_and(w, jnp.int32(-65536)), jnp.float32)


def _bf16_lo(w):
    return plsc.bitcast(lax.shift_left(w, 16), jnp.float32)


def _zero_f32(buf, nwords):
    @plsc.parallel_loop(0, nwords, 16, unroll=4)
    def zb(i):
        buf[pl.ds(i, 16)] = jnp.zeros((16,), jnp.float32)


@functools.partial(
    pl.kernel,
    out_type=jax.ShapeDtypeStruct((_NW * _N,), jnp.float32),
    mesh=_mesh,
    scratch_types=[
        pltpu.VMEM((_N,), jnp.float32),
        pltpu.VMEM((_EP,), jnp.int32),
        pltpu.VMEM((_EP,), jnp.float32),
    ],
    compiler_params=_sc_params,
)
def _sc_deg(col_hbm, ew_hbm, out_hbm, dacc, cb, wb):
    wid = lax.axis_index("c") * _NS + lax.axis_index("s")
    base = wid * _EP
    pltpu.sync_copy(col_hbm.at[pl.ds(base, _EP)], cb)
    pltpu.sync_copy(ew_hbm.at[pl.ds(base, _EP)], wb)
    _zero_f32(dacc, _N)

    @plsc.parallel_loop(0, _EP, 16, unroll=8)
    def body(i):
        s = pl.ds(i, 16)
        plsc.addupdate_scatter(dacc, [cb[s]], wb[s])
    pltpu.sync_copy(dacc, out_hbm.at[pl.ds(wid * _N, _N)])


_EP2 = _E // 16


@functools.partial(
    pl.kernel,
    out_type=(
        jax.ShapeDtypeStruct((_E // 2,), jnp.int32),
        jax.ShapeDtypeStruct((_E,), jnp.int32),
    ),
    mesh=_mesh,
    scratch_types=[
        pltpu.VMEM((_N,), jnp.float32),
        pltpu.VMEM((_EP2,), jnp.int32),
        pltpu.VMEM((_EP2,), jnp.int32),
        pltpu.VMEM((_EP2,), jnp.float32),
        pltpu.VMEM((_EP2 // 2,), jnp.int32),
        pltpu.VMEM((_EP2,), jnp.int32),
    ],
    compiler_params=_sc_params,
)
def _sc_norm(row_hbm, col_hbm, ew_hbm, dis_hbm, nrm_hbm, pk_hbm,
             disv, rb, cb, wb, on, op):
    cid = lax.axis_index("c")
    sid = lax.axis_index("s")
    wid = cid * _NS + sid

    @pl.when(wid < 16)
    def _():
        base = wid * _EP2
        pltpu.sync_copy(dis_hbm, disv)
        pltpu.sync_copy(row_hbm.at[pl.ds(base, _EP2)], rb)
        pltpu.sync_copy(col_hbm.at[pl.ds(base, _EP2)], cb)
        pltpu.sync_copy(ew_hbm.at[pl.ds(base, _EP2)], wb)

        @plsc.parallel_loop(0, _EP2, 32, unroll=8)
        def body(i):
            sa = pl.ds(i, 16)
            sb = pl.ds(i + 16, 16)
            ra = rb[sa]
            ca = cb[sa]
            na = plsc.load_gather(disv, [ra]) * wb[sa] \
                * plsc.load_gather(disv, [ca])
            rbb = rb[sb]
            cbb = cb[sb]
            nb_ = plsc.load_gather(disv, [rbb]) * wb[sb] \
                * plsc.load_gather(disv, [cbb])
            on[pl.ds(pl.multiple_of(lax.shift_right_logical(i, 1), 16), 16)] \
                = _bf16_pack_pair(na, nb_)
            op[sa] = lax.shift_left(ra, 14) + ca
            op[sb] = lax.shift_left(rbb, 14) + cbb
        pltpu.sync_copy(on, nrm_hbm.at[pl.ds(wid * (_EP2 // 2), _EP2 // 2)])
        pltpu.sync_copy(op, pk_hbm.at[pl.ds(base, _EP2)])


@functools.partial(
    pl.kernel,
    out_type=jax.ShapeDtypeStruct((_H * _N,), jnp.float32),
    mesh=_mesh,
    scratch_types=[
        pltpu.VMEM((2 * _N,), jnp.float32),
        pltpu.VMEM((2 * _N,), jnp.float32),
        pltpu.VMEM((2 * _N,), jnp.float32),
        pltpu.VMEM((_N,), jnp.int32),
        pltpu.VMEM((_CE,), jnp.int32),
        pltpu.VMEM((_CE,), jnp.int32),
        pltpu.VMEM((_CE // 2,), jnp.int32),
        pltpu.VMEM((_CE // 2,), jnp.int32),
        pltpu.VMEM((128,), jnp.float32),
        pltpu.SemaphoreType.DMA,
        pltpu.SemaphoreType.DMA,
        pltpu.SemaphoreType.DMA,
        pltpu.SemaphoreType.DMA,
    ],
    compiler_params=_sc_params,
)
def _sc_layer(h_hbm, pk_hbm, nm_hbm, cf_hbm, out_hbm,
              tx0, tx1, acc, packb, pb0, pb1, nb0, nb1, cfs,
              sp0, sp1, sn0, sn1):

    wid = lax.axis_index("c") * _NS + lax.axis_index("s")
    hbase = wid * (2 * _N)
    pltpu.sync_copy(cf_hbm, cfs)
    pltpu.sync_copy(h_hbm.at[pl.ds(hbase, 2 * _N)], tx0)

    c0v = cfs[pl.ds(0, 16)] + 1.0
    tx0_hi = tx0.at[pl.ds(_N, _N)]
    tx1_hi = tx1.at[pl.ds(_N, _N)]
    acc_hi = acc.at[pl.ds(_N, _N)]

    @plsc.parallel_loop(0, _N, 16, unroll=4)
    def init_acc(i):
        s = pl.ds(i, 16)
        v0 = tx0[s]
        v1 = tx0_hi[s]
        acc[s] = v0 * c0v
        acc_hi[s] = v1 * c0v
        packb[s] = _bf16_pack_pair(v0, v1)
        tx1[s] = jnp.zeros((16,), jnp.float32)
        tx1_hi[s] = jnp.zeros((16,), jnp.float32)

    pbufs = (pb0, pb1)
    nbufs = (nb0, nb1)
    psems = (sp0, sp1)
    nsems = (sn0, sn1)

    def start_chunk(c, par):
        pltpu.make_async_copy(
            pk_hbm.at[pl.ds(c * _CE, _CE)], pbufs[par], psems[par]).start()
        pltpu.make_async_copy(
            nm_hbm.at[pl.ds(c * (_CE // 2), _CE // 2)],
            nbufs[par], nsems[par]).start()

    def wait_chunk(c, par):
        pltpu.make_async_copy(
            pk_hbm.at[pl.ds(c * _CE, _CE)], pbufs[par], psems[par]).wait()
        pltpu.make_async_copy(
            nm_hbm.at[pl.ds(c * (_CE // 2), _CE // 2)],
            nbufs[par], nsems[par]).wait()

    for k in range(1, _P + 1):
        cur = tx0 if k % 2 == 1 else tx1
        nxt = tx1 if k % 2 == 1 else tx0
        cur_hi = cur.at[pl.ds(_N, _N)]
        nxt_hi = nxt.at[pl.ds(_N, _N)]
        start_chunk(0, 0)

        def grp(pb, nb, base):
            nw = nb[pl.ds(pl.multiple_of(lax.shift_right_logical(base, 1), 16),
                          16)]
            for half in (0, 1):
                s = pl.ds(base + 16 * half, 16)
                p16 = pb[s]
                nm16 = _bf16_hi(nw) if half == 0 else _bf16_lo(nw)
                c16 = jnp.bitwise_and(p16, 16383)
                r16 = lax.shift_right_logical(p16, 14)
                w = plsc.load_gather(packb, [c16])
                plsc.addupdate_scatter(nxt, [r16], _bf16_hi(w) * nm16)
                plsc.addupdate_scatter(nxt_hi, [r16], _bf16_lo(w) * nm16)

        def chunk_pair(j, c):
            for par in (0, 1):
                ch = 2 * j + par

                @pl.when(ch + 1 < _NCH)
                def _():
                    start_chunk(ch + 1, 1 - par)

                wait_chunk(ch, par)

                @plsc.parallel_loop(0, _CE, 32, unroll=5)
                def inner(base):
                    grp(pbufs[par], nbufs[par], base)
            return c
        lax.fori_loop(0, _NCH // 2, chunk_pair, 0)

        ckv = cfs[pl.ds(k * 16, 16)]

        @plsc.parallel_loop(0, _N, 16, unroll=4)
        def upd_acc(i):
            s = pl.ds(i, 16)
            v0 = nxt[s]
            v1 = nxt_hi[s]
            acc[s] = acc[s] + v0 * ckv
            acc_hi[s] = acc_hi[s] + v1 * ckv
            if k < _P:
                packb[s] = _bf16_pack_pair(v0, v1)
                cur[s] = jnp.zeros((16,), jnp.float32)
                cur_hi[s] = jnp.zeros((16,), jnp.float32)

    pltpu.sync_copy(acc, out_hbm.at[pl.ds(hbase, 2 * _N)])


def _make_sc_edgez(eh):
    nch = eh // _CE

    @functools.partial(
        pl.kernel,
        out_type=jax.ShapeDtypeStruct(((_H // 2) * eh,), jnp.int32),
        mesh=_mesh,
        scratch_types=[
            pltpu.VMEM((2 * _N,), jnp.float32),
            pltpu.VMEM((2 * _N,), jnp.float32),
            pltpu.VMEM((_N,), jnp.int32),
            pltpu.VMEM((_N,), jnp.int32),
            pltpu.VMEM((_CE,), jnp.int32),
            pltpu.VMEM((_CE,), jnp.int32),
            pltpu.VMEM((_CE,), jnp.int32),
            pltpu.VMEM((_CE,), jnp.int32),
            pltpu.SemaphoreType.DMA,
            pltpu.SemaphoreType.DMA,
            pltpu.SemaphoreType.DMA,
            pltpu.SemaphoreType.DMA,
        ],
        compiler_params=_sc_params,
    )
    def _sc_edgez(wr_hbm, wc_hbm, pk_hbm, z_hbm,
                  wrv, wcv, wrp, wcp, pb0, pb1, zb0, zb1,
                  sp0, sp1, so0, so1):
        wid = lax.axis_index("c") * _NS + lax.axis_index("s")
        fr0 = 2 * wid
        pltpu.sync_copy(wr_hbm.at[pl.ds(fr0 * _N, 2 * _N)], wrv)
        pltpu.sync_copy(wc_hbm.at[pl.ds(fr0 * _N, 2 * _N)], wcv)
        wrv_hi = wrv.at[pl.ds(_N, _N)]
        wcv_hi = wcv.at[pl.ds(_N, _N)]

        @plsc.parallel_loop(0, _N, 16, unroll=4)
        def pack_src(i):
            s = pl.ds(i, 16)
            wrp[s] = _bf16_pack_pair(wrv[s], wrv_hi[s])
            wcp[s] = _bf16_pack_pair(wcv[s], wcv_hi[s])

        pbufs = (pb0, pb1)
        zbufs = (zb0, zb1)
        psems = (sp0, sp1)
        osems = (so0, so1)

        def start_in(c, par):
            pltpu.make_async_copy(
                pk_hbm.at[pl.ds(c * _CE, _CE)], pbufs[par], psems[par]).start()

        def wait_in(c, par):
            pltpu.make_async_copy(
                pk_hbm.at[pl.ds(c * _CE, _CE)], pbufs[par], psems[par]).wait()

        def start_out(c, par):
            pltpu.make_async_copy(
                zbufs[par],
                z_hbm.at[pl.ds(wid * eh + c * _CE, _CE)], osems[par]).start()

        def wait_out(c, par):
            pltpu.make_async_copy(
                zbufs[par],
                z_hbm.at[pl.ds(wid * eh + c * _CE, _CE)], osems[par]).wait()

        start_in(0, 0)

        def chunk_pair(j, c):
            for par in (0, 1):
                ch = 2 * j + par

                @pl.when(ch + 1 < nch)
                def _():
                    start_in(ch + 1, 1 - par)

                wait_in(ch, par)
                @pl.when(ch >= 2)
                def _():
                    wait_out(ch - 2, par)

                pb = pbufs[par]
                zb = zbufs[par]

                @plsc.parallel_loop(0, _CE, 16, unroll=10)
                def inner(base):
                    s = pl.ds(base, 16)
                    p16 = pb[s]
                    c16 = jnp.bitwise_and(p16, 16383)
                    r16 = lax.shift_right_logical(p16, 14)
                    gr = plsc.load_gather(wrp, [r16])
                    gc = plsc.load_gather(wcp, [c16])
                    f0 = _bf16_hi(gr) + _bf16_hi(gc)
                    f1 = _bf16_lo(gr) + _bf16_lo(gc)
                    zb[s] = _bf16_pack_pair(f0, f1)
                start_out(ch, par)
            return c
        lax.fori_loop(0, nch // 2, chunk_pair, 0)
        wait_out(nch - 2, 0)
        wait_out(nch - 1, 1)

    return _sc_edgez


_sc_edgez_half = _make_sc_edgez(_E // 2)


def _stats_coeffs(hT, cw1a, cw1b, cb1l, cw2p, cb2p):
    xm = jnp.mean(hT, axis=1, keepdims=True)
    m = jnp.mean(hT)
    sq = jnp.sum(hT * hT)
    mm = jnp.float32(_N * _H)
    var1 = (sq - mm * m * m) / (mm - 1.0)
    st = jnp.sqrt(var1)
    stats = jnp.concatenate(
        [jnp.full((1, 1), 1.0, jnp.float32) * m,
         jnp.full((1, 1), 1.0, jnp.float32) * st,
         jnp.full((1, 1), float(_N), jnp.float32),
         jnp.full((1, 1), float(_E), jnp.float32)], axis=0)
    hid = (lax.dot_general(xm, cw1a, (((0,), (0,)), ((), ())))
           + lax.dot_general(stats, cw1b, (((0,), (0,)), ((), ())))
           + cb1l)
    hid = jnp.maximum(hid, 0.0)
    lo = lax.dot_general(hid, cw2p, (((1,), (0,)), ((), ()))) + cb2p
    z = lo - jnp.max(lo, axis=1, keepdims=True)
    ez = jnp.exp(z)
    return ez / jnp.sum(ez, axis=1, keepdims=True)


def _tc_prep_body(x_ref, w_ref, b_ref, dp_ref, cw1a, cw1b, cb1l, cw2p, cb2p,
                  hT_ref, dis_ref, cf_ref):
    hT = lax.dot_general(w_ref[...], x_ref[...], (((0,), (1,)), ((), ())))
    hT = hT + b_ref[...]
    hT_ref[...] = hT
    deg = jnp.sum(dp_ref[...], axis=0, keepdims=True)
    dis = jnp.minimum(lax.rsqrt(deg), 1e6)
    dis = jnp.where(jnp.isinf(dis), 0.0, dis)
    dis_ref[...] = dis
    cf_ref[...] = _stats_coeffs(hT, cw1a[...], cw1b[...], cb1l[...],
                                cw2p[...], cb2p[...])


_tc_prep = pl.pallas_call(
    _tc_prep_body,
    out_shape=(
        jax.ShapeDtypeStruct((_H, _N), jnp.float32),
        jax.ShapeDtypeStruct((1, _N), jnp.float32),
        jax.ShapeDtypeStruct((1, 8), jnp.float32),
    ),
)


def _ln(hrT, g, b):
    mu = jnp.mean(hrT, axis=0, keepdims=True)
    d = hrT - mu
    var = jnp.mean(d * d, axis=0, keepdims=True)
    return d * lax.rsqrt(var + 1e-5) * g + b


def _tc_inter_body(hr_ref, g_ref, b_ref, cw1a, cw1b, cb1l, cw2p, cb2p,
                   hn_ref, cf_ref):
    hn = _ln(hr_ref[...], g_ref[...], b_ref[...])
    hn_ref[...] = hn
    cf_ref[...] = _stats_coeffs(hn, cw1a[...], cw1b[...], cb1l[...],
                                cw2p[...], cb2p[...])


_tc_inter = pl.pallas_call(
    _tc_inter_body,
    out_shape=(
        jax.ShapeDtypeStruct((_H, _N), jnp.float32),
        jax.ShapeDtypeStruct((1, 8), jnp.float32),
    ),
)


def _tc_final_body(hr_ref, g_ref, b_ref, w1a_ref, w1b_ref,
                   hn_ref, wr_ref, wc_ref):
    hn = _ln(hr_ref[...], g_ref[...], b_ref[...])
    hn_ref[...] = hn
    wr_ref[...] = lax.dot_general(w1a_ref[...], hn, (((0,), (0,)), ((), ())))
    wc_ref[...] = lax.dot_general(w1b_ref[...], hn, (((0,), (0,)), ((), ())))


_tc_final = pl.pallas_call(
    _tc_final_body,
    out_shape=(
        jax.ShapeDtypeStruct((_H, _N), jnp.float32),
        jax.ShapeDtypeStruct((_H, _N), jnp.float32),
        jax.ShapeDtypeStruct((_H, _N), jnp.float32),
    ),
)

_EB = 3200


def _tc_edge_body(z_ref, eb1e_ref, eb1o_ref, w2e_ref, w2o_ref, eb2_ref,
                  w3_ref, eb3_ref, out_ref):
    w = z_ref[...]
    ze = lax.bitcast_convert_type(
        jnp.bitwise_and(w, jnp.int32(-65536)), jnp.float32)
    zo = lax.bitcast_convert_type(
        lax.shift_left(w, 16), jnp.float32)
    a1e = jnp.maximum(ze + eb1e_ref[...], 0.0)
    a1o = jnp.maximum(zo + eb1o_ref[...], 0.0)
    a2 = (lax.dot_general(w2e_ref[...], a1e, (((0,), (0,)), ((), ())))
          + lax.dot_general(w2o_ref[...], a1o, (((0,), (0,)), ((), ()))))
    a2 = jnp.maximum(a2 + eb2_ref[...], 0.0)
    s = lax.dot_general(w3_ref[...], a2, (((0,), (0,)), ((), ())))
    s = s + eb3_ref[...]
    out_ref[...] = 1.0 / (1.0 + jnp.exp(-s))


def _make_tc_edge(eh):
    return pl.pallas_call(
        _tc_edge_body,
        grid=(eh // _EB,),
        in_specs=[
            pl.BlockSpec((_H // 2, _EB), lambda i: (0, i)),
            pl.BlockSpec((_H // 2, 1), lambda i: (0, 0)),
            pl.BlockSpec((_H // 2, 1), lambda i: (0, 0)),
            pl.BlockSpec((32, 32), lambda i: (0, 0)),
            pl.BlockSpec((32, 32), lambda i: (0, 0)),
            pl.BlockSpec((32, 1), lambda i: (0, 0)),
            pl.BlockSpec((32, 1), lambda i: (0, 0)),
            pl.BlockSpec((1, 1), lambda i: (0, 0)),
        ],
        out_specs=pl.BlockSpec((1, _EB), lambda i: (0, i)),
        out_shape=jax.ShapeDtypeStruct((1, eh), jnp.float32),
    )


_tc_edge_half = _make_tc_edge(_E // 2)


def kernel(x, edge_index, edge_weight, in_w, in_b, cw1, cb1, cw2, cb2,
           ln_g, ln_b, ew1, eb1, ew2, eb2, ew3, eb3):
    L = cw1.shape[0]
    row = edge_index[0]
    col = edge_index[1]

    in_b2 = in_b.reshape(_H, 1)
    cw1a = [cw1[l, :_H, :] for l in range(L)]
    cw1b = [cw1[l, _H:, :] for l in range(L)]
    cb1l = [cb1[l].reshape(1, -1) for l in range(L)]
    cw2p = [jnp.pad(cw2[l], ((0, 0), (0, 8 - (_P + 1)))) for l in range(L)]
    cb2p = [jnp.pad(cb2[l], (0, 8 - (_P + 1)),
                    constant_values=-1e30).reshape(1, 8) for l in range(L)]

    deg_parts = _sc_deg(col, edge_weight).reshape(_NW, _N)
    hT, dis, cf = _tc_prep(x, in_w, in_b2, deg_parts,
                           cw1a[0], cw1b[0], cb1l[0], cw2p[0], cb2p[0])
    nrm, packed = _sc_norm(row, col, edge_weight, dis.reshape(_N))

    for l in range(L):
        cf_rep = jnp.broadcast_to(cf.reshape(8)[:, None], (8, 16)).reshape(128)
        hr_flat = _sc_layer(hT.reshape(-1), packed, nrm, cf_rep)
        hrT = hr_flat.reshape(_H, _N)
        if l < L - 1:
            hT, cf = _tc_inter(hrT, ln_g[l].reshape(_H, 1),
                               ln_b[l].reshape(_H, 1),
                               cw1a[l + 1], cw1b[l + 1], cb1l[l + 1],
                               cw2p[l + 1], cb2p[l + 1])
        else:
            hT, wrT, wcT = _tc_final(hrT, ln_g[l].reshape(_H, 1),
                                     ln_b[l].reshape(_H, 1),
                                     ew1[:_H, :], ew1[_H:, :])

    wr_flat = wrT.reshape(-1)
    wc_flat = wcT.reshape(-1)
    eh = _E // 2
    mlp_args = (eb1[0::2].reshape(_H // 2, 1),
                eb1[1::2].reshape(_H // 2, 1),
                ew2[0::2, :], ew2[1::2, :],
                eb2.reshape(32, 1),
                ew3, eb3.reshape(1, 1))
    zw_a = _sc_edgez_half(wr_flat, wc_flat, packed[:eh])
    preds_a = _tc_edge_half(zw_a.reshape(_H // 2, eh), *mlp_args)
    zw_b = _sc_edgez_half(wr_flat, wc_flat, packed[eh:])
    preds_b = _tc_edge_half(zw_b.reshape(_H // 2, eh), *mlp_args)
    preds = jnp.concatenate([preds_a.reshape(eh), preds_b.reshape(eh)])
    return preds, hT.T
